# baseline (speedup 1.0000x reference)
import numpy as np
import jax
import jax.numpy as jnp
from jax.experimental import pallas as pl
from jax.experimental.pallas import tpu as pltpu

_B, _K, _M = 16384, 26, 128
_ROWS = 512


def _hist_block(sim, lab):
    mask = (lab > 0.5).astype(jnp.float32)
    mx = jnp.max(sim, axis=2, keepdims=True)
    row = jax.lax.broadcasted_iota(jnp.int32, (1, 1, _M), 2).astype(jnp.float32)
    iota = jnp.broadcast_to(row, sim.shape)
    idx = jnp.min(jnp.where(sim == mx, iota, jnp.float32(_M)), axis=2, keepdims=True)
    return jnp.sum(jnp.where(iota == idx, mask[:, :, None], 0.0), axis=0)


def _balance_kernel(sim0_ref, sim1_ref, lab0_ref, lab1_ref, out_ref, acc_ref):
    i = pl.program_id(0)
    n = pl.num_programs(0)
    partial = _hist_block(sim0_ref[...], lab0_ref[...])
    partial += _hist_block(sim1_ref[...], lab1_ref[...])

    @pl.when(i == 0)
    def _init():
        acc_ref[...] = jnp.zeros_like(acc_ref)

    acc_ref[...] += partial

    @pl.when(i == n - 1)
    def _finish():
        counts = acc_ref[...]
        tot = jnp.sum(counts, axis=1, keepdims=True)
        dist = counts / (tot + 1e-8)
        ent = -jnp.sum(dist * jnp.log(dist + 1e-8), axis=1, keepdims=True)
        max_ent = np.float32(np.log(_M))
        loss_k = (max_ent - ent) / max_ent
        has = (tot > 0).astype(jnp.float32)
        total_loss = jnp.sum(loss_k * has)
        num = jnp.sum(has)
        out_ref[0, 0] = jnp.where(num > 0, total_loss / jnp.maximum(num, 1.0), 0.0)


def kernel(similarities, concept_labels):
    B_, K_, M_ = similarities.shape
    rows = min(_ROWS, B_ // 2)
    half = B_ // (2 * rows)
    out = pl.pallas_call(
        _balance_kernel,
        grid=(half,),
        in_specs=[
            pl.BlockSpec((rows, K_, M_), lambda i: (i, 0, 0)),
            pl.BlockSpec((rows, K_, M_), lambda i, h=half: (i + h, 0, 0)),
            pl.BlockSpec((rows, K_), lambda i: (i, 0)),
            pl.BlockSpec((rows, K_), lambda i, h=half: (i + h, 0)),
        ],
        out_specs=pl.BlockSpec(memory_space=pltpu.SMEM),
        out_shape=jax.ShapeDtypeStruct((1, 1), jnp.float32),
        scratch_shapes=[pltpu.VMEM((K_, M_), jnp.float32)],
    )(similarities, similarities, concept_labels, concept_labels)
    return out[0, 0]


# speedup vs baseline: 1.0030x; 1.0030x over previous
import numpy as np
import jax
import jax.numpy as jnp
from jax.experimental import pallas as pl
from jax.experimental.pallas import tpu as pltpu

_B, _K, _M = 16384, 26, 128
_ROWS = 256


def _balance_kernel(sim_ref, lab_ref, out_ref, acc_ref):
    i = pl.program_id(0)
    n = pl.num_programs(0)
    sim = sim_ref[...]
    lab = lab_ref[...]
    mask = (lab > 0.5).astype(jnp.float32)
    mx = jnp.max(sim, axis=2, keepdims=True)
    row = jax.lax.broadcasted_iota(jnp.int32, (1, 1, _M), 2).astype(jnp.float32)
    iota = jnp.broadcast_to(row, sim.shape)
    idx = jnp.min(jnp.where(sim == mx, iota, jnp.float32(_M)), axis=2, keepdims=True)
    partial = jnp.sum(jnp.where(iota == idx, mask[:, :, None], 0.0), axis=0)

    @pl.when(i == 0)
    def _init():
        acc_ref[...] = jnp.zeros_like(acc_ref)

    acc_ref[...] += partial

    @pl.when(i == n - 1)
    def _finish():
        counts = acc_ref[...]
        tot = jnp.sum(counts, axis=1, keepdims=True)
        dist = counts / (tot + 1e-8)
        ent = -jnp.sum(dist * jnp.log(dist + 1e-8), axis=1, keepdims=True)
        max_ent = np.float32(np.log(_M))
        loss_k = (max_ent - ent) / max_ent
        has = (tot > 0).astype(jnp.float32)
        total_loss = jnp.sum(loss_k * has)
        num = jnp.sum(has)
        out_ref[0, 0] = jnp.where(num > 0, total_loss / jnp.maximum(num, 1.0), 0.0)


def kernel(similarities, concept_labels):
    B_, K_, M_ = similarities.shape
    rows = min(_ROWS, B_)
    grid = (B_ // rows,)
    out = pl.pallas_call(
        _balance_kernel,
        grid=grid,
        in_specs=[
            pl.BlockSpec((rows, K_, M_), lambda i: (i, 0, 0)),
            pl.BlockSpec((rows, K_), lambda i: (i, 0)),
        ],
        out_specs=pl.BlockSpec(memory_space=pltpu.SMEM),
        out_shape=jax.ShapeDtypeStruct((1, 1), jnp.float32),
        scratch_shapes=[pltpu.VMEM((K_, M_), jnp.float32)],
    )(similarities, concept_labels)
    return out[0, 0]


# speedup vs baseline: 1.2027x; 1.1992x over previous
import numpy as np
import jax
import jax.numpy as jnp
from jax.experimental import pallas as pl
from jax.experimental.pallas import tpu as pltpu

_B, _K, _M = 16384, 26, 128
_ROWS = 256


def _balance_kernel(sim_ref, lab_ref, out_ref, acc_ref):
    i = pl.program_id(0)
    n = pl.num_programs(0)
    sim = sim_ref[...]
    lab = lab_ref[...]
    mask = (lab > 0.5).astype(jnp.float32)
    idx = jnp.argmax(sim, axis=2)[:, :, None]
    iota = jax.lax.broadcasted_iota(jnp.int32, sim.shape, 2)
    partial = jnp.sum(jnp.where(iota == idx, mask[:, :, None], 0.0), axis=0)

    @pl.when(i == 0)
    def _init():
        acc_ref[...] = jnp.zeros_like(acc_ref)

    acc_ref[...] += partial

    @pl.when(i == n - 1)
    def _finish():
        counts = acc_ref[...]
        tot = jnp.sum(counts, axis=1, keepdims=True)
        dist = counts / (tot + 1e-8)
        ent = -jnp.sum(dist * jnp.log(dist + 1e-8), axis=1, keepdims=True)
        max_ent = np.float32(np.log(_M))
        loss_k = (max_ent - ent) / max_ent
        has = (tot > 0).astype(jnp.float32)
        total_loss = jnp.sum(loss_k * has)
        num = jnp.sum(has)
        out_ref[0, 0] = jnp.where(num > 0, total_loss / jnp.maximum(num, 1.0), 0.0)


def kernel(similarities, concept_labels):
    B_, K_, M_ = similarities.shape
    rows = min(_ROWS, B_)
    grid = (B_ // rows,)
    out = pl.pallas_call(
        _balance_kernel,
        grid=grid,
        in_specs=[
            pl.BlockSpec((rows, K_, M_), lambda i: (i, 0, 0)),
            pl.BlockSpec((rows, K_), lambda i: (i, 0)),
        ],
        out_specs=pl.BlockSpec(memory_space=pltpu.SMEM),
        out_shape=jax.ShapeDtypeStruct((1, 1), jnp.float32),
        scratch_shapes=[pltpu.VMEM((K_, M_), jnp.float32)],
    )(similarities, concept_labels)
    return out[0, 0]


# speedup vs baseline: 1.2128x; 1.0084x over previous
import numpy as np
import jax
import jax.numpy as jnp
from jax.experimental import pallas as pl
from jax.experimental.pallas import tpu as pltpu

_B, _K, _M = 16384, 26, 128
_ROWS = 512


def _balance_kernel(sim_ref, lab_ref, out_ref, acc_ref):
    i = pl.program_id(0)
    n = pl.num_programs(0)
    sim = sim_ref[...]
    lab = lab_ref[...]
    mask = (lab > 0.5).astype(jnp.float32)
    idx = jnp.argmax(sim, axis=2)[:, :, None]
    iota = jax.lax.broadcasted_iota(jnp.int32, sim.shape, 2)
    partial = jnp.sum(jnp.where(iota == idx, mask[:, :, None], 0.0), axis=0)

    @pl.when(i == 0)
    def _init():
        acc_ref[...] = jnp.zeros_like(acc_ref)

    acc_ref[...] += partial

    @pl.when(i == n - 1)
    def _finish():
        counts = acc_ref[...]
        tot = jnp.sum(counts, axis=1, keepdims=True)
        dist = counts / (tot + 1e-8)
        ent = -jnp.sum(dist * jnp.log(dist + 1e-8), axis=1, keepdims=True)
        max_ent = np.float32(np.log(_M))
        loss_k = (max_ent - ent) / max_ent
        has = (tot > 0).astype(jnp.float32)
        total_loss = jnp.sum(loss_k * has)
        num = jnp.sum(has)
        out_ref[0, 0] = jnp.where(num > 0, total_loss / jnp.maximum(num, 1.0), 0.0)


def kernel(similarities, concept_labels):
    B_, K_, M_ = similarities.shape
    rows = min(_ROWS, B_)
    grid = (B_ // rows,)
    out = pl.pallas_call(
        _balance_kernel,
        grid=grid,
        in_specs=[
            pl.BlockSpec((rows, K_, M_), lambda i: (i, 0, 0)),
            pl.BlockSpec((rows, K_), lambda i: (i, 0)),
        ],
        out_specs=pl.BlockSpec(memory_space=pltpu.SMEM),
        out_shape=jax.ShapeDtypeStruct((1, 1), jnp.float32),
        scratch_shapes=[pltpu.VMEM((K_, M_), jnp.float32)],
    )(similarities, concept_labels)
    return out[0, 0]
